# baseline (speedup 1.0000x reference)
import jax
import jax.numpy as jnp
from jax import lax
from jax.experimental import pallas as pl
from jax.experimental.pallas import tpu as pltpu, tpu_sc as plsc

B, S, H = 4, 8192, 1024
N = B * S
NC, NS = 2, 16
NW = NC * NS
PER_W = N // NW
CHUNK = 32
NCHUNK = PER_W // CHUNK
NBUF = 3


def _emb_body(text_table, feat_table, text_ids, feat_ids,
              text_out, feat_out, idx_v,
              rows0, rows1, rows2, g0, g1, g2, s0, s1, s2):
    wid = lax.axis_index("s") * NC + lax.axis_index("c")
    base = wid * PER_W
    rows = (rows0, rows1, rows2)
    gsem = (g0, g1, g2)
    ssem = (s0, s1, s2)
    pltpu.sync_copy(text_ids.at[pl.ds(base, PER_W)], idx_v.at[pl.ds(0, PER_W)])
    pltpu.sync_copy(feat_ids.at[pl.ds(base, PER_W)],
                    idx_v.at[pl.ds(PER_W, PER_W)])

    for t, (table, out) in enumerate(((text_table, text_out),
                                      (feat_table, feat_out))):
        def start_gather(ch, b):
            idx_s = idx_v.at[pl.ds(t * PER_W + ch * CHUNK, CHUNK)]
            pltpu.async_copy(table.at[idx_s], rows[b], gsem[b])

        def wait_gather(b):
            pltpu.make_async_copy(table.at[pl.ds(0, CHUNK)], rows[b],
                                  gsem[b]).wait()

        def start_scatter(ch, b):
            pltpu.async_copy(rows[b],
                             out.at[pl.ds(base + ch * CHUNK, CHUNK)], ssem[b])

        def wait_scatter(b):
            pltpu.make_async_copy(rows[b], out.at[pl.ds(0, CHUNK)],
                                  ssem[b]).wait()

        def steady(j, jm3):
            b = jm3
            wait_gather(b)
            start_scatter(j, b)
            bn = (jm3 + 2) % 3
            wait_scatter(bn)
            start_gather(j + 2, bn)

        start_gather(0, 0)
        start_gather(1, 1)
        wait_gather(0)
        start_scatter(0, 0)
        start_gather(2, 2)
        steady(1, 1)
        steady(2, 2)

        @pl.loop(3, 30, step=3)
        def _(c):
            for k in range(3):
                steady(c + k, k)

        wait_gather(0)
        start_scatter(30, 0)
        wait_gather(1)
        start_scatter(31, 1)
        wait_scatter(2)
        wait_scatter(0)
        wait_scatter(1)


def kernel(input_ids, feature_ids, text_table, feature_table):
    t_ids = input_ids.reshape(-1).astype(jnp.int32)
    f_ids = feature_ids.reshape(-1).astype(jnp.int32)
    mesh = plsc.VectorSubcoreMesh(core_axis_name="c", subcore_axis_name="s")
    fn = pl.kernel(
        _emb_body,
        out_type=(jax.ShapeDtypeStruct((N, H), jnp.float32),
                  jax.ShapeDtypeStruct((N, H), jnp.float32)),
        mesh=mesh,
        scratch_types=[
            pltpu.VMEM((2 * PER_W,), jnp.int32),
            pltpu.VMEM((CHUNK, H), jnp.float32),
            pltpu.VMEM((CHUNK, H), jnp.float32),
            pltpu.VMEM((CHUNK, H), jnp.float32),
            pltpu.SemaphoreType.DMA,
            pltpu.SemaphoreType.DMA,
            pltpu.SemaphoreType.DMA,
            pltpu.SemaphoreType.DMA,
            pltpu.SemaphoreType.DMA,
            pltpu.SemaphoreType.DMA,
        ],
    )
    t_out, f_out = fn(text_table, feature_table, t_ids, f_ids)
    return t_out.reshape(B, S, H), f_out.reshape(B, S, H)
